# Initial kernel scaffold; baseline (speedup 1.0000x reference)
import jax
import jax.numpy as jnp
from jax.experimental import pallas as pl


def kernel(x):
    raise NotImplementedError("write your pallas kernel here")


# speedup vs baseline: 2.4968x; 2.4968x over previous
import jax
import jax.numpy as jnp
from jax import lax
from jax.experimental import pallas as pl
from jax.experimental.pallas import tpu as pltpu

_B, _C, _H, _W = 8, 96, 224, 224
_HW = _H * _W
_CORR = 3.0 / 64.0


def _rcp_body(x_ref, o_ref):
    xb = x_ref[0]

    ch = jnp.sum(xb, axis=1, keepdims=True)
    ch_row = ch.reshape(1, _C)

    row = lax.broadcasted_iota(jnp.int32, (_C, _C), 0)
    col = lax.broadcasted_iota(jnp.int32, (_C, _C), 1)
    a = jnp.broadcast_to(ch, (_C, _C))
    b = jnp.broadcast_to(ch_row, (_C, _C))
    less = (b < a) | ((b == a) & (col < row))
    rank = jnp.sum(less.astype(jnp.float32), axis=1, keepdims=True)

    ocol = lax.broadcasted_iota(jnp.int32, (_C, 8), 1)
    cidx = lax.broadcasted_iota(jnp.int32, (_C, 8), 0).astype(jnp.float32)
    key = jnp.where(ocol < 2, jnp.broadcast_to(rank, (_C, 8)), cidx)
    thresh = jnp.where(ocol % 2 == 0, 32.0, 64.0)
    base = jnp.where(ocol % 2 == 0, 1.0 / 64.0, 1.0 / 32.0)
    ind = (key < thresh).astype(jnp.float32)
    wt = jnp.where(ocol < 4, base - _CORR * ind, 0.0)

    rcp = lax.dot_general(
        wt, xb, (((0,), (0,)), ((), ())),
        preferred_element_type=jnp.float32,
    )

    rmin = jnp.min(rcp, axis=1, keepdims=True)
    rmax = jnp.max(rcp, axis=1, keepdims=True)
    norm = (rcp - rmin) / (rmax - rmin + 1e-8)
    o_ref[0] = norm[0:4]


def kernel(x):
    xr = x.reshape(_B, _C, _HW)
    out = pl.pallas_call(
        _rcp_body,
        grid=(_B,),
        in_specs=[pl.BlockSpec((1, _C, _HW), lambda b: (b, 0, 0))],
        out_specs=pl.BlockSpec((1, 4, _HW), lambda b: (b, 0, 0)),
        out_shape=jax.ShapeDtypeStruct((_B, 4, _HW), jnp.float32),
        compiler_params=pltpu.CompilerParams(
            dimension_semantics=("arbitrary",),
            vmem_limit_bytes=110 * 1024 * 1024,
        ),
    )(xr)
    return out.reshape(_B, 4, _H, _W)
